# baseline (speedup 1.0000x reference)
import functools

import jax
import jax.numpy as jnp
from jax import lax
from jax.experimental import pallas as pl
from jax.experimental.pallas import tpu as pltpu
from jax.experimental.pallas import tpu_sc as plsc

B, SIZE, VOCAB, EMBED, HIDDEN = 1024, 50, 100000, 64, 512
NIDX = B * SIZE

NC, NS = 2, 16
NW = NC * NS
ROWS_PER_W = NIDX // NW

VTILE = 2000
NT = VOCAB // VTILE
TPC = NT // 2
K_STREAMS = 5
SUBR = VTILE // K_STREAMS


def _sc_gather(table, idx):
    mesh = plsc.VectorSubcoreMesh(core_axis_name="c", subcore_axis_name="s")

    @functools.partial(
        pl.kernel,
        out_type=jax.ShapeDtypeStruct((NIDX, EMBED), jnp.float32),
        mesh=mesh,
        scratch_types=[
            pltpu.VMEM((ROWS_PER_W,), jnp.int32),
            pltpu.VMEM((ROWS_PER_W, EMBED), jnp.float32),
            pltpu.SemaphoreType.DMA,
        ],
        compiler_params=pltpu.CompilerParams(use_tc_tiling_on_sc=False),
    )
    def gather_kernel(table_hbm, idx_hbm, out_hbm, idx_v, rows_v, sem):
        wid = lax.axis_index("s") * NC + lax.axis_index("c")
        base = wid * ROWS_PER_W
        pltpu.sync_copy(idx_hbm.at[pl.ds(base, ROWS_PER_W)], idx_v)
        pltpu.async_copy(table_hbm.at[idx_v], rows_v, sem).wait()
        pltpu.sync_copy(rows_v, out_hbm.at[pl.ds(base, ROWS_PER_W)])

    return gather_kernel(table, idx)


def _mlp_manual(flat, W1, b1_2d, b2t, W2t):
    mesh = pltpu.create_tensorcore_mesh("core", num_cores=2)

    @functools.partial(
        pl.kernel,
        out_type=jax.ShapeDtypeStruct((VOCAB, B), jnp.float32),
        mesh=mesh,
        scratch_types=[
            pltpu.VMEM((B, SIZE * EMBED), jnp.float32),
            pltpu.VMEM((SIZE * EMBED, HIDDEN), jnp.float32),
            pltpu.VMEM((1, HIDDEN), jnp.float32),
            pltpu.VMEM((HIDDEN, B), jnp.bfloat16),
            pltpu.VMEM((VTILE, NT), jnp.float32),
            pltpu.VMEM((3, VTILE, HIDDEN), jnp.float32),
            pltpu.VMEM((2, VTILE, B), jnp.float32),
            pltpu.SemaphoreType.DMA((3, K_STREAMS)),
            pltpu.SemaphoreType.DMA((2, K_STREAMS)),
        ],
    )
    def mlp_kernel(flat_hbm, w1_hbm, b1_hbm, b2t_hbm, w2t_hbm, out_hbm,
                   flat_v, w1_v, b1_v, ht_v, b2t_v, w2_buf, out_buf,
                   in_sems, out_sems):
        c = lax.axis_index("core")

        def in_copy(tt, k):
            row = tt * VTILE + k * SUBR
            return pltpu.make_async_copy(
                w2t_hbm.at[pl.ds(row, SUBR), :],
                w2_buf.at[lax.rem(tt, 3), pl.ds(k * SUBR, SUBR), :],
                in_sems.at[lax.rem(tt, 3), k],
            )

        def start_in(tt):
            for k in range(K_STREAMS):
                in_copy(tt, k).start()

        def out_copy(oslot, tt, k):
            row = tt * VTILE + k * SUBR
            return pltpu.make_async_copy(
                out_buf.at[oslot, pl.ds(k * SUBR, SUBR), :],
                out_hbm.at[pl.ds(row, SUBR), :],
                out_sems.at[oslot, k],
            )

        t0 = c * TPC
        start_in(t0)
        start_in(t0 + 1)
        pltpu.sync_copy(flat_hbm, flat_v)
        pltpu.sync_copy(w1_hbm, w1_v)
        pltpu.sync_copy(b1_hbm, b1_v)
        pltpu.sync_copy(b2t_hbm, b2t_v)

        acc1 = jnp.dot(
            flat_v[...].astype(jnp.bfloat16),
            w1_v[...].astype(jnp.bfloat16),
            preferred_element_type=jnp.float32,
        )
        ht_v[...] = jnp.maximum(acc1 + b1_v[...], 0.0).T.astype(jnp.bfloat16)

        @pl.loop(0, TPC)
        def _(j):
            t = t0 + j

            @pl.when(j < TPC - 2)
            def _():
                start_in(t + 2)

            for k in range(K_STREAMS):
                in_copy(t, k).wait()

            @pl.when(j >= 2)
            def _():
                for k in range(K_STREAMS):
                    out_copy(lax.rem(j, 2), t - 2, k).wait()

            w2v = w2_buf[lax.rem(t, 3)].astype(jnp.bfloat16)
            acc = jnp.dot(w2v, ht_v[...], preferred_element_type=jnp.float32)
            lane = jax.lax.broadcasted_iota(jnp.int32, (VTILE, NT), 1)
            b2_col = jnp.sum(
                jnp.where(lane == t, b2t_v[...], 0.0), axis=1, keepdims=True
            )
            out_buf[lax.rem(j, 2)] = acc + b2_col

            for k in range(K_STREAMS):
                out_copy(lax.rem(j, 2), t, k).start()

        t_last = t0 + TPC - 1
        for k in range(K_STREAMS):
            out_copy(0, t_last - 1, k).wait()
        for k in range(K_STREAMS):
            out_copy(1, t_last, k).wait()

    return mlp_kernel(flat, W1, b1_2d, b2t, W2t)


def kernel(x, embed, W1, b1, W2, b2):
    idx = x.reshape(-1).astype(jnp.int32)
    flat_rows = _sc_gather(embed, idx)
    flat = flat_rows.reshape(B, SIZE * EMBED)

    b1_2d = b1.reshape(1, HIDDEN)
    b2t = b2.reshape(NT, VTILE).T
    W2t = W2.T

    out_t = _mlp_manual(flat, W1, b1_2d, b2t, W2t)
    return out_t.T


# speedup vs baseline: 1.0421x; 1.0421x over previous
import functools

import jax
import jax.numpy as jnp
from jax import lax
from jax.experimental import pallas as pl
from jax.experimental.pallas import tpu as pltpu
from jax.experimental.pallas import tpu_sc as plsc

B, SIZE, VOCAB, EMBED, HIDDEN = 1024, 50, 100000, 64, 512
NIDX = B * SIZE

NC, NS = 2, 16
NW = NC * NS
ROWS_PER_W = NIDX // NW

VTILE = 2000
NT = VOCAB // VTILE
TPC = NT // 2
K_STREAMS = 5
SUBR = VTILE // K_STREAMS


def _sc_gather(table, idx):
    mesh = plsc.VectorSubcoreMesh(core_axis_name="c", subcore_axis_name="s")

    @functools.partial(
        pl.kernel,
        out_type=jax.ShapeDtypeStruct((NIDX, EMBED), jnp.float32),
        mesh=mesh,
        scratch_types=[
            pltpu.VMEM((ROWS_PER_W,), jnp.int32),
            pltpu.VMEM((ROWS_PER_W, EMBED), jnp.float32),
            pltpu.SemaphoreType.DMA,
        ],
        compiler_params=pltpu.CompilerParams(use_tc_tiling_on_sc=False),
    )
    def gather_kernel(table_hbm, idx_hbm, out_hbm, idx_v, rows_v, sem):
        wid = lax.axis_index("s") * NC + lax.axis_index("c")
        base = wid * ROWS_PER_W
        pltpu.sync_copy(idx_hbm.at[pl.ds(base, ROWS_PER_W)], idx_v)
        pltpu.async_copy(table_hbm.at[idx_v], rows_v, sem).wait()
        pltpu.sync_copy(rows_v, out_hbm.at[pl.ds(base, ROWS_PER_W)])

    return gather_kernel(table, idx)


def _mm1_body(flat_ref, w1_ref, b1_ref, ht_ref):
    acc = jnp.dot(
        flat_ref[...].astype(jnp.bfloat16),
        w1_ref[...].astype(jnp.bfloat16),
        preferred_element_type=jnp.float32,
    )
    ht_ref[...] = jnp.maximum(acc + b1_ref[...], 0.0).T.astype(jnp.bfloat16)


def _mm2_manual(ht, b2t, W2t):
    mesh = pltpu.create_tensorcore_mesh("core", num_cores=2)

    @functools.partial(
        pl.kernel,
        out_type=jax.ShapeDtypeStruct((VOCAB, B), jnp.float32),
        mesh=mesh,
        scratch_types=[
            pltpu.VMEM((HIDDEN, B), jnp.bfloat16),
            pltpu.VMEM((VTILE, NT), jnp.float32),
            pltpu.VMEM((3, VTILE, HIDDEN), jnp.float32),
            pltpu.VMEM((2, VTILE, B), jnp.float32),
            pltpu.SemaphoreType.DMA((3, K_STREAMS)),
            pltpu.SemaphoreType.DMA((2, K_STREAMS)),
        ],
    )
    def mm2_kernel(ht_hbm, b2t_hbm, w2t_hbm, out_hbm,
                   ht_v, b2t_v, w2_buf, out_buf, in_sems, out_sems):
        c = lax.axis_index("core")

        def in_copy(tt, k):
            row = tt * VTILE + k * SUBR
            return pltpu.make_async_copy(
                w2t_hbm.at[pl.ds(row, SUBR), :],
                w2_buf.at[lax.rem(tt, 3), pl.ds(k * SUBR, SUBR), :],
                in_sems.at[lax.rem(tt, 3), k],
            )

        def start_in(tt):
            for k in range(K_STREAMS):
                in_copy(tt, k).start()

        def out_copy(oslot, tt, k):
            row = tt * VTILE + k * SUBR
            return pltpu.make_async_copy(
                out_buf.at[oslot, pl.ds(k * SUBR, SUBR), :],
                out_hbm.at[pl.ds(row, SUBR), :],
                out_sems.at[oslot, k],
            )

        t0 = c * TPC
        start_in(t0)
        start_in(t0 + 1)
        pltpu.sync_copy(ht_hbm, ht_v)
        pltpu.sync_copy(b2t_hbm, b2t_v)

        @pl.loop(0, TPC)
        def _(j):
            t = t0 + j

            @pl.when(j < TPC - 2)
            def _():
                start_in(t + 2)

            for k in range(K_STREAMS):
                in_copy(t, k).wait()

            @pl.when(j >= 2)
            def _():
                for k in range(K_STREAMS):
                    out_copy(lax.rem(j, 2), t - 2, k).wait()

            w2v = w2_buf[lax.rem(t, 3)].astype(jnp.bfloat16)
            acc = jnp.dot(w2v, ht_v[...], preferred_element_type=jnp.float32)
            lane = jax.lax.broadcasted_iota(jnp.int32, (VTILE, NT), 1)
            b2_col = jnp.sum(
                jnp.where(lane == t, b2t_v[...], 0.0), axis=1, keepdims=True
            )
            out_buf[lax.rem(j, 2)] = acc + b2_col

            for k in range(K_STREAMS):
                out_copy(lax.rem(j, 2), t, k).start()

        t_last = t0 + TPC - 1
        for k in range(K_STREAMS):
            out_copy(0, t_last - 1, k).wait()
        for k in range(K_STREAMS):
            out_copy(1, t_last, k).wait()

    return mm2_kernel(ht, b2t, W2t)


def kernel(x, embed, W1, b1, W2, b2):
    idx = x.reshape(-1).astype(jnp.int32)
    flat_rows = _sc_gather(embed, idx)
    flat = flat_rows.reshape(B, SIZE * EMBED)

    b1_2d = b1.reshape(1, HIDDEN)
    b2t = b2.reshape(NT, VTILE).T
    W2t = W2.T

    ht = pl.pallas_call(
        _mm1_body,
        grid=(2,),
        in_specs=[
            pl.BlockSpec((B // 2, SIZE * EMBED), lambda i: (i, 0)),
            pl.BlockSpec((SIZE * EMBED, HIDDEN), lambda i: (0, 0)),
            pl.BlockSpec((1, HIDDEN), lambda i: (0, 0)),
        ],
        out_specs=pl.BlockSpec((HIDDEN, B // 2), lambda i: (0, i)),
        out_shape=jax.ShapeDtypeStruct((HIDDEN, B), jnp.bfloat16),
        compiler_params=pltpu.CompilerParams(
            dimension_semantics=("parallel",),
        ),
    )(flat, W1, b1_2d)

    out_t = _mm2_manual(ht, b2t, W2t)
    return out_t.T


# speedup vs baseline: 1.0499x; 1.0075x over previous
import functools

import jax
import jax.numpy as jnp
from jax import lax
from jax.experimental import pallas as pl
from jax.experimental.pallas import tpu as pltpu
from jax.experimental.pallas import tpu_sc as plsc

B, SIZE, VOCAB, EMBED, HIDDEN = 1024, 50, 100000, 64, 512
NIDX = B * SIZE

NC, NS = 2, 16
NW = NC * NS
ROWS_PER_W = NIDX // NW

VTILE = 2000
NT = VOCAB // VTILE
TPC = NT // 2
K_STREAMS = 10
SUBR = VTILE // K_STREAMS


def _sc_gather(table, idx):
    mesh = plsc.VectorSubcoreMesh(core_axis_name="c", subcore_axis_name="s")

    @functools.partial(
        pl.kernel,
        out_type=jax.ShapeDtypeStruct((NIDX, EMBED), jnp.float32),
        mesh=mesh,
        scratch_types=[
            pltpu.VMEM((ROWS_PER_W,), jnp.int32),
            pltpu.VMEM((ROWS_PER_W, EMBED), jnp.float32),
            pltpu.SemaphoreType.DMA,
        ],
        compiler_params=pltpu.CompilerParams(use_tc_tiling_on_sc=False),
    )
    def gather_kernel(table_hbm, idx_hbm, out_hbm, idx_v, rows_v, sem):
        wid = lax.axis_index("s") * NC + lax.axis_index("c")
        base = wid * ROWS_PER_W
        pltpu.sync_copy(idx_hbm.at[pl.ds(base, ROWS_PER_W)], idx_v)
        pltpu.async_copy(table_hbm.at[idx_v], rows_v, sem).wait()
        pltpu.sync_copy(rows_v, out_hbm.at[pl.ds(base, ROWS_PER_W)])

    return gather_kernel(table, idx)


def _mm1_body(flat_ref, w1_ref, b1_ref, ht_ref):
    acc = jnp.dot(
        flat_ref[...].astype(jnp.bfloat16),
        w1_ref[...].astype(jnp.bfloat16),
        preferred_element_type=jnp.float32,
    )
    ht_ref[...] = jnp.maximum(acc + b1_ref[...], 0.0).T.astype(jnp.bfloat16)


def _mm2_manual(ht, b2t, W2t):
    mesh = pltpu.create_tensorcore_mesh("core", num_cores=2)

    @functools.partial(
        pl.kernel,
        out_type=jax.ShapeDtypeStruct((VOCAB, B), jnp.float32),
        mesh=mesh,
        scratch_types=[
            pltpu.VMEM((HIDDEN, B), jnp.bfloat16),
            pltpu.VMEM((VTILE, NT), jnp.float32),
            pltpu.VMEM((3, VTILE, HIDDEN), jnp.float32),
            pltpu.VMEM((3, VTILE, B), jnp.float32),
            pltpu.SemaphoreType.DMA((3, K_STREAMS)),
            pltpu.SemaphoreType.DMA((3, K_STREAMS)),
        ],
    )
    def mm2_kernel(ht_hbm, b2t_hbm, w2t_hbm, out_hbm,
                   ht_v, b2t_v, w2_buf, out_buf, in_sems, out_sems):
        c = lax.axis_index("core")

        def in_copy(tt, k):
            row = tt * VTILE + k * SUBR
            return pltpu.make_async_copy(
                w2t_hbm.at[pl.ds(row, SUBR), :],
                w2_buf.at[lax.rem(tt, 3), pl.ds(k * SUBR, SUBR), :],
                in_sems.at[lax.rem(tt, 3), k],
            )

        def start_in(tt):
            for k in range(K_STREAMS):
                in_copy(tt, k).start()

        def out_copy(oslot, tt, k):
            row = tt * VTILE + k * SUBR
            return pltpu.make_async_copy(
                out_buf.at[oslot, pl.ds(k * SUBR, SUBR), :],
                out_hbm.at[pl.ds(row, SUBR), :],
                out_sems.at[oslot, k],
            )

        t0 = c * TPC
        start_in(t0)
        start_in(t0 + 1)
        pltpu.sync_copy(ht_hbm, ht_v)
        pltpu.sync_copy(b2t_hbm, b2t_v)

        @pl.loop(0, TPC)
        def _(j):
            t = t0 + j

            @pl.when(j < TPC - 2)
            def _():
                start_in(t + 2)

            for k in range(K_STREAMS):
                in_copy(t, k).wait()

            @pl.when(j >= 3)
            def _():
                for k in range(K_STREAMS):
                    out_copy(lax.rem(j, 3), t - 3, k).wait()

            w2v = w2_buf[lax.rem(t, 3)].astype(jnp.bfloat16)
            acc = jnp.dot(w2v, ht_v[...], preferred_element_type=jnp.float32)
            lane = jax.lax.broadcasted_iota(jnp.int32, (VTILE, NT), 1)
            b2_col = jnp.sum(
                jnp.where(lane == t, b2t_v[...], 0.0), axis=1, keepdims=True
            )
            out_buf[lax.rem(j, 3)] = acc + b2_col

            for k in range(K_STREAMS):
                out_copy(lax.rem(j, 3), t, k).start()

        t_last = t0 + TPC - 1
        for m in (2, 1, 0):
            for k in range(K_STREAMS):
                out_copy(lax.rem(TPC - 1 - m, 3), t_last - m, k).wait()

    return mm2_kernel(ht, b2t, W2t)


def kernel(x, embed, W1, b1, W2, b2):
    idx = x.reshape(-1).astype(jnp.int32)
    flat_rows = _sc_gather(embed, idx)
    flat = flat_rows.reshape(B, SIZE * EMBED)

    b1_2d = b1.reshape(1, HIDDEN)
    b2t = b2.reshape(NT, VTILE).T
    W2t = W2.T

    ht = pl.pallas_call(
        _mm1_body,
        grid=(2,),
        in_specs=[
            pl.BlockSpec((B // 2, SIZE * EMBED), lambda i: (i, 0)),
            pl.BlockSpec((SIZE * EMBED, HIDDEN), lambda i: (0, 0)),
            pl.BlockSpec((1, HIDDEN), lambda i: (0, 0)),
        ],
        out_specs=pl.BlockSpec((HIDDEN, B // 2), lambda i: (0, i)),
        out_shape=jax.ShapeDtypeStruct((HIDDEN, B), jnp.bfloat16),
        compiler_params=pltpu.CompilerParams(
            dimension_semantics=("parallel",),
        ),
    )(flat, W1, b1_2d)

    out_t = _mm2_manual(ht, b2t, W2t)
    return out_t.T
